# Initial kernel scaffold; baseline (speedup 1.0000x reference)
import jax
import jax.numpy as jnp
from jax.experimental import pallas as pl


def kernel(x, edge_index, batch, W1a, b1a, g1, be1, W1b, b1b, W2a, b2a, g2, be2, W2b, b2b, W3a, b3a, g3, be3, W3b, b3b, Wh1, bh1, Wh2, bh2):
    raise NotImplementedError("write your pallas kernel here")


# speedup vs baseline: 5.7432x; 5.7432x over previous
import functools

import jax
import jax.numpy as jnp
from jax import lax
from jax.experimental import pallas as pl
from jax.experimental.pallas import tpu as pltpu
from jax.experimental.pallas import tpu_sc as plsc

_N = 10000
_E = 320000
_D = 128
_G = 64

_NS = 16
_EPT = _E // _NS
_CH = 80
_GC = 25
_NG = _EPT // (_CH * _GC)
_NPAD = 10112
_RPW = _NPAD // _NS


def _edge_agg_body(h_hbm, src_hbm, dst_hbm, out_hbm,
                   src_v, dst_v, buf_a, buf_b, acc, sem_a, sem_b):
    sid = lax.axis_index("s")

    @pl.loop(0, _CH)
    def _zrow(r):
        @pl.loop(0, _D, step=16)
        def _zcol(c):
            buf_a[r, pl.ds(c, 16)] = jnp.zeros((16,), jnp.float32)

    @pl.loop(0, _RPW // _CH)
    def _zcopy(i):
        pltpu.sync_copy(buf_a, acc.at[pl.ds(sid * _RPW + i * _CH, _CH)])

    pltpu.sync_copy(
        buf_a.at[pl.ds(0, _RPW % _CH)],
        acc.at[pl.ds(sid * _RPW + (_RPW // _CH) * _CH, _RPW % _CH)])

    plsc.subcore_barrier()

    def _start(c, buf, sem):
        pltpu.async_copy(h_hbm.at[src_v.at[c]], buf, sem)

    def _wait(buf, sem):
        pltpu.make_async_copy(h_hbm.at[src_v.at[0]], buf, sem).wait()

    def _scat(c, buf):
        pltpu.sync_copy(buf, acc.at[dst_v.at[c]], add=True)

    @pl.loop(0, _NG)
    def _group(g):
        pltpu.sync_copy(src_hbm.at[sid, g], src_v)
        pltpu.sync_copy(dst_hbm.at[sid, g], dst_v)

        _start(0, buf_a, sem_a)

        @pl.loop(0, _GC - 1, step=2)
        def _pair(c):
            _start(c + 1, buf_b, sem_b)
            _wait(buf_a, sem_a)
            _scat(c, buf_a)
            _start(c + 2, buf_a, sem_a)
            _wait(buf_b, sem_b)
            _scat(c + 1, buf_b)

        _wait(buf_a, sem_a)
        _scat(_GC - 1, buf_a)

    plsc.subcore_barrier()

    pltpu.sync_copy(acc.at[pl.ds(sid * _RPW, _RPW)],
                    out_hbm.at[pl.ds(sid * _RPW, _RPW)])


@functools.cache
def _edge_agg_kernel():
    mesh = plsc.VectorSubcoreMesh(core_axis_name="c", subcore_axis_name="s",
                                  num_cores=1, num_subcores=_NS)
    return pl.kernel(
        _edge_agg_body,
        out_type=jax.ShapeDtypeStruct((_NPAD, _D), jnp.float32),
        mesh=mesh,
        scratch_types=[
            pltpu.VMEM((_GC, _CH), jnp.int32),
            pltpu.VMEM((_GC, _CH), jnp.int32),
            pltpu.VMEM((_CH, _D), jnp.float32),
            pltpu.VMEM((_CH, _D), jnp.float32),
            pltpu.VMEM_SHARED((_NPAD, _D), jnp.float32),
            pltpu.SemaphoreType.DMA,
            pltpu.SemaphoreType.DMA,
        ],
    )


def _edge_agg(h, src4, dst4):
    return _edge_agg_kernel()(h, src4, dst4)


def _dot(a, b):
    return lax.dot_general(a, b, (((1,), (0,)), ((), ())),
                           precision=lax.Precision.HIGHEST,
                           preferred_element_type=jnp.float32)


def _gin_math(x_ref, agg_ref, wa_ref, ba_ref, g_ref, be_ref, wb_ref, bb_ref):
    h0 = x_ref[...] + agg_ref[:_N, :]
    t = _dot(h0, wa_ref[...]) + ba_ref[...]
    m = jnp.mean(t, axis=0, keepdims=True)
    tc = t - m
    v = jnp.mean(tc * tc, axis=0, keepdims=True)
    t = tc * lax.rsqrt(v + 1e-5) * g_ref[...] + be_ref[...]
    t = jnp.maximum(t, 0.0)
    return _dot(t, wb_ref[...]) + bb_ref[...]


def _gin_dense_body(tanh, x_ref, agg_ref, wa_ref, ba_ref, g_ref, be_ref,
                    wb_ref, bb_ref, o_ref):
    h1 = _gin_math(x_ref, agg_ref, wa_ref, ba_ref, g_ref, be_ref,
                   wb_ref, bb_ref)
    if tanh:
        h1 = jnp.tanh(h1)
    o_ref[...] = jnp.maximum(h1, 0.0)


def _gin_dense(x, agg, wa, ba, g, be, wb, bb, tanh):
    return pl.pallas_call(
        functools.partial(_gin_dense_body, tanh),
        out_shape=jax.ShapeDtypeStruct((_N, _D), jnp.float32),
    )(x, agg, wa, ba, g, be, wb, bb)


def _final_body(x_ref, agg_ref, wa_ref, ba_ref, g_ref, be_ref, wb_ref,
                bb_ref, batch_ref, wh1_ref, bh1_ref, wh2_ref, bh2_ref,
                o_ref):
    h1 = jnp.tanh(_gin_math(x_ref, agg_ref, wa_ref, ba_ref, g_ref, be_ref,
                            wb_ref, bb_ref))
    seg = lax.broadcasted_iota(jnp.int32, (_N, _G), 1)
    onehot = (batch_ref[...] == seg).astype(jnp.float32)
    sums = lax.dot_general(onehot, h1, (((0,), (0,)), ((), ())),
                           precision=lax.Precision.HIGHEST,
                           preferred_element_type=jnp.float32)
    cnt = jnp.sum(onehot, axis=0)
    pooled = sums / jnp.maximum(cnt, 1.0)[:, None]
    p = jnp.maximum(_dot(pooled, wh1_ref[...]) + bh1_ref[...], 0.0)
    o_ref[...] = _dot(p, wh2_ref[...]) + bh2_ref[...]


def _final(x, agg, wa, ba, g, be, wb, bb, batch2d, wh1, bh1, wh2, bh2):
    return pl.pallas_call(
        _final_body,
        out_shape=jax.ShapeDtypeStruct((_G, wh2.shape[1]), jnp.float32),
    )(x, agg, wa, ba, g, be, wb, bb, batch2d, wh1, bh1, wh2, bh2)


def kernel(x, edge_index, batch, W1a, b1a, g1, be1, W1b, b1b, W2a, b2a, g2,
           be2, W2b, b2b, W3a, b3a, g3, be3, W3b, b3b, Wh1, bh1, Wh2, bh2):
    src4 = edge_index[0].reshape(_NS, _NG, _GC, _CH)
    dst4 = edge_index[1].reshape(_NS, _NG, _GC, _CH)
    batch2d = batch.reshape(_N, 1)

    agg = _edge_agg(x, src4, dst4)
    h = _gin_dense(x, agg, W1a, b1a, g1, be1, W1b, b1b, False)
    agg = _edge_agg(h, src4, dst4)
    h = _gin_dense(h, agg, W2a, b2a, g2, be2, W2b, b2b, True)
    agg = _edge_agg(h, src4, dst4)
    return _final(h, agg, W3a, b3a, g3, be3, W3b, b3b, batch2d,
                  Wh1, bh1, Wh2, bh2)


# speedup vs baseline: 6.6929x; 1.1654x over previous
import functools

import jax
import jax.numpy as jnp
from jax import lax
from jax.experimental import pallas as pl
from jax.experimental.pallas import tpu as pltpu
from jax.experimental.pallas import tpu_sc as plsc

_N = 10000
_E = 320000
_D = 128
_G = 64

_NS = 16
_EPT = _E // _NS
_CH = 80
_GC = 25
_NG = _EPT // (_CH * _GC)
_NPAD = 10112
_RPW = _NPAD // _NS


def _edge_agg_body(h_hbm, src_hbm, dst_hbm, out_hbm,
                   src_v, dst_v, buf_a, buf_b, buf_c, acc,
                   sem_a, sem_b, sem_c):
    sid = lax.axis_index("s")

    @pl.loop(0, _CH)
    def _zrow(r):
        @pl.loop(0, _D, step=16)
        def _zcol(c):
            buf_a[r, pl.ds(c, 16)] = jnp.zeros((16,), jnp.float32)

    @pl.loop(0, _RPW // _CH)
    def _zcopy(i):
        pltpu.sync_copy(buf_a, acc.at[pl.ds(sid * _RPW + i * _CH, _CH)])

    pltpu.sync_copy(
        buf_a.at[pl.ds(0, _RPW % _CH)],
        acc.at[pl.ds(sid * _RPW + (_RPW // _CH) * _CH, _RPW % _CH)])

    plsc.subcore_barrier()

    def _start(c, buf, sem):
        pltpu.async_copy(h_hbm.at[src_v.at[c]], buf, sem)

    def _wait(buf, sem):
        pltpu.make_async_copy(h_hbm.at[src_v.at[0]], buf, sem).wait()

    def _scat(c, buf):
        pltpu.sync_copy(buf, acc.at[dst_v.at[c]], add=True)

    @pl.loop(0, _NG)
    def _group(g):
        pltpu.sync_copy(src_hbm.at[sid, g], src_v)
        pltpu.sync_copy(dst_hbm.at[sid, g], dst_v)

        _start(0, buf_a, sem_a)
        _start(1, buf_b, sem_b)

        @pl.loop(0, _GC - 6, step=3)
        def _trip(c):
            _start(c + 2, buf_c, sem_c)
            _wait(buf_a, sem_a)
            _scat(c, buf_a)
            _start(c + 3, buf_a, sem_a)
            _wait(buf_b, sem_b)
            _scat(c + 1, buf_b)
            _start(c + 4, buf_b, sem_b)
            _wait(buf_c, sem_c)
            _scat(c + 2, buf_c)

        _start(_GC - 2, buf_c, sem_c)
        _wait(buf_a, sem_a)
        _scat(_GC - 4, buf_a)
        _start(_GC - 1, buf_a, sem_a)
        _wait(buf_b, sem_b)
        _scat(_GC - 3, buf_b)
        _wait(buf_c, sem_c)
        _scat(_GC - 2, buf_c)
        _wait(buf_a, sem_a)
        _scat(_GC - 1, buf_a)

    plsc.subcore_barrier()

    pltpu.sync_copy(acc.at[pl.ds(sid * _RPW, _RPW)],
                    out_hbm.at[pl.ds(sid * _RPW, _RPW)])


@functools.cache
def _edge_agg_kernel():
    mesh = plsc.VectorSubcoreMesh(core_axis_name="c", subcore_axis_name="s",
                                  num_cores=1, num_subcores=_NS)
    return pl.kernel(
        _edge_agg_body,
        out_type=jax.ShapeDtypeStruct((_NPAD, _D), jnp.float32),
        mesh=mesh,
        scratch_types=[
            pltpu.VMEM((_GC, _CH), jnp.int32),
            pltpu.VMEM((_GC, _CH), jnp.int32),
            pltpu.VMEM((_CH, _D), jnp.float32),
            pltpu.VMEM((_CH, _D), jnp.float32),
            pltpu.VMEM((_CH, _D), jnp.float32),
            pltpu.VMEM_SHARED((_NPAD, _D), jnp.float32),
            pltpu.SemaphoreType.DMA,
            pltpu.SemaphoreType.DMA,
            pltpu.SemaphoreType.DMA,
        ],
    )


def _edge_agg(h, src4, dst4):
    return _edge_agg_kernel()(h, src4, dst4)


def _dot(a, b):
    return lax.dot_general(a, b, (((1,), (0,)), ((), ())),
                           precision=lax.Precision.HIGHEST,
                           preferred_element_type=jnp.float32)


def _gin_math(x_ref, agg_ref, wa_ref, ba_ref, g_ref, be_ref, wb_ref, bb_ref):
    h0 = x_ref[...] + agg_ref[:_N, :]
    t = _dot(h0, wa_ref[...]) + ba_ref[...]
    m = jnp.mean(t, axis=0, keepdims=True)
    tc = t - m
    v = jnp.mean(tc * tc, axis=0, keepdims=True)
    t = tc * lax.rsqrt(v + 1e-5) * g_ref[...] + be_ref[...]
    t = jnp.maximum(t, 0.0)
    return _dot(t, wb_ref[...]) + bb_ref[...]


def _gin_dense_body(tanh, x_ref, agg_ref, wa_ref, ba_ref, g_ref, be_ref,
                    wb_ref, bb_ref, o_ref):
    h1 = _gin_math(x_ref, agg_ref, wa_ref, ba_ref, g_ref, be_ref,
                   wb_ref, bb_ref)
    if tanh:
        h1 = jnp.tanh(h1)
    o_ref[...] = jnp.maximum(h1, 0.0)


def _gin_dense(x, agg, wa, ba, g, be, wb, bb, tanh):
    return pl.pallas_call(
        functools.partial(_gin_dense_body, tanh),
        out_shape=jax.ShapeDtypeStruct((_N, _D), jnp.float32),
    )(x, agg, wa, ba, g, be, wb, bb)


def _final_body(x_ref, agg_ref, wa_ref, ba_ref, g_ref, be_ref, wb_ref,
                bb_ref, batch_ref, wh1_ref, bh1_ref, wh2_ref, bh2_ref,
                o_ref):
    h1 = jnp.tanh(_gin_math(x_ref, agg_ref, wa_ref, ba_ref, g_ref, be_ref,
                            wb_ref, bb_ref))
    seg = lax.broadcasted_iota(jnp.int32, (_N, _G), 1)
    onehot = (batch_ref[...] == seg).astype(jnp.float32)
    sums = lax.dot_general(onehot, h1, (((0,), (0,)), ((), ())),
                           precision=lax.Precision.HIGHEST,
                           preferred_element_type=jnp.float32)
    cnt = jnp.sum(onehot, axis=0)
    pooled = sums / jnp.maximum(cnt, 1.0)[:, None]
    p = jnp.maximum(_dot(pooled, wh1_ref[...]) + bh1_ref[...], 0.0)
    o_ref[...] = _dot(p, wh2_ref[...]) + bh2_ref[...]


def _final(x, agg, wa, ba, g, be, wb, bb, batch2d, wh1, bh1, wh2, bh2):
    return pl.pallas_call(
        _final_body,
        out_shape=jax.ShapeDtypeStruct((_G, wh2.shape[1]), jnp.float32),
    )(x, agg, wa, ba, g, be, wb, bb, batch2d, wh1, bh1, wh2, bh2)


def kernel(x, edge_index, batch, W1a, b1a, g1, be1, W1b, b1b, W2a, b2a, g2,
           be2, W2b, b2b, W3a, b3a, g3, be3, W3b, b3b, Wh1, bh1, Wh2, bh2):
    src4 = edge_index[0].reshape(_NS, _NG, _GC, _CH)
    dst4 = edge_index[1].reshape(_NS, _NG, _GC, _CH)
    batch2d = batch.reshape(_N, 1)

    agg = _edge_agg(x, src4, dst4)
    h = _gin_dense(x, agg, W1a, b1a, g1, be1, W1b, b1b, False)
    agg = _edge_agg(h, src4, dst4)
    h = _gin_dense(h, agg, W2a, b2a, g2, be2, W2b, b2b, True)
    agg = _edge_agg(h, src4, dst4)
    return _final(h, agg, W3a, b3a, g3, be3, W3b, b3b, batch2d,
                  Wh1, bh1, Wh2, bh2)


# speedup vs baseline: 7.1709x; 1.0714x over previous
import functools

import jax
import jax.numpy as jnp
from jax import lax
from jax.experimental import pallas as pl
from jax.experimental.pallas import tpu as pltpu
from jax.experimental.pallas import tpu_sc as plsc

_N = 10000
_E = 320000
_D = 128
_G = 64

_NS = 16
_EPT = _E // _NS
_CH = 80
_GC = 25
_NG = _EPT // (_CH * _GC)
_NPAD = 10112
_RPW = _NPAD // _NS


def _edge_agg_body(h_hbm, src_hbm, dst_hbm, out_hbm,
                   src_v, dst_v, buf_a, buf_b, buf_c, acc,
                   sem_a, sem_b, sem_c):
    sid = lax.axis_index("s")

    @pl.loop(0, _CH)
    def _zrow(r):
        @pl.loop(0, _D, step=16)
        def _zcol(c):
            buf_a[r, pl.ds(c, 16)] = jnp.zeros((16,), jnp.float32)

    @pl.loop(0, _RPW // _CH)
    def _zcopy(i):
        pltpu.sync_copy(buf_a, acc.at[pl.ds(sid * _RPW + i * _CH, _CH)])

    pltpu.sync_copy(
        buf_a.at[pl.ds(0, _RPW % _CH)],
        acc.at[pl.ds(sid * _RPW + (_RPW // _CH) * _CH, _RPW % _CH)])

    plsc.subcore_barrier()

    def _start(c, buf, sem):
        pltpu.async_copy(h_hbm.at[src_v.at[c]], buf, sem)

    def _wait(buf, sem):
        pltpu.make_async_copy(h_hbm.at[src_v.at[0]], buf, sem).wait()

    def _scat(c, buf):
        pltpu.sync_copy(buf, acc.at[dst_v.at[c]], add=True)

    @pl.loop(0, _NG)
    def _group(g):
        pltpu.sync_copy(src_hbm.at[sid, g], src_v)
        pltpu.sync_copy(dst_hbm.at[sid, g], dst_v)

        _start(0, buf_a, sem_a)
        _start(1, buf_b, sem_b)

        @pl.loop(0, _GC - 6, step=3)
        def _trip(c):
            _start(c + 2, buf_c, sem_c)
            _wait(buf_a, sem_a)
            _scat(c, buf_a)
            _start(c + 3, buf_a, sem_a)
            _wait(buf_b, sem_b)
            _scat(c + 1, buf_b)
            _start(c + 4, buf_b, sem_b)
            _wait(buf_c, sem_c)
            _scat(c + 2, buf_c)

        _start(_GC - 2, buf_c, sem_c)
        _wait(buf_a, sem_a)
        _scat(_GC - 4, buf_a)
        _start(_GC - 1, buf_a, sem_a)
        _wait(buf_b, sem_b)
        _scat(_GC - 3, buf_b)
        _wait(buf_c, sem_c)
        _scat(_GC - 2, buf_c)
        _wait(buf_a, sem_a)
        _scat(_GC - 1, buf_a)

    plsc.subcore_barrier()

    pltpu.sync_copy(acc.at[pl.ds(sid * _RPW, _RPW)],
                    out_hbm.at[pl.ds(sid * _RPW, _RPW)])


@functools.cache
def _edge_agg_kernel():
    mesh = plsc.VectorSubcoreMesh(core_axis_name="c", subcore_axis_name="s",
                                  num_cores=1, num_subcores=_NS)
    return pl.kernel(
        _edge_agg_body,
        out_type=jax.ShapeDtypeStruct((_NPAD, _D), jnp.float32),
        mesh=mesh,
        scratch_types=[
            pltpu.VMEM((_GC, _CH), jnp.int32),
            pltpu.VMEM((_GC, _CH), jnp.int32),
            pltpu.VMEM((_CH, _D), jnp.float32),
            pltpu.VMEM((_CH, _D), jnp.float32),
            pltpu.VMEM((_CH, _D), jnp.float32),
            pltpu.VMEM_SHARED((_NPAD, _D), jnp.float32),
            pltpu.SemaphoreType.DMA,
            pltpu.SemaphoreType.DMA,
            pltpu.SemaphoreType.DMA,
        ],
    )


def _edge_agg(h, src4, dst4):
    return _edge_agg_kernel()(h, src4, dst4)


def _dot(a, b):
    return lax.dot_general(a, b, (((1,), (0,)), ((), ())),
                           precision=lax.Precision.DEFAULT,
                           preferred_element_type=jnp.float32)


def _gin_math(x_ref, agg_ref, wa_ref, ba_ref, g_ref, be_ref, wb_ref, bb_ref):
    h0 = x_ref[...] + agg_ref[:_N, :]
    t = _dot(h0, wa_ref[...]) + ba_ref[...]
    m = jnp.mean(t, axis=0, keepdims=True)
    tc = t - m
    v = jnp.mean(tc * tc, axis=0, keepdims=True)
    t = tc * lax.rsqrt(v + 1e-5) * g_ref[...] + be_ref[...]
    t = jnp.maximum(t, 0.0)
    return _dot(t, wb_ref[...]) + bb_ref[...]


def _gin_dense_body(tanh, x_ref, agg_ref, wa_ref, ba_ref, g_ref, be_ref,
                    wb_ref, bb_ref, o_ref):
    h1 = _gin_math(x_ref, agg_ref, wa_ref, ba_ref, g_ref, be_ref,
                   wb_ref, bb_ref)
    if tanh:
        h1 = jnp.tanh(h1)
    o_ref[...] = jnp.maximum(h1, 0.0)


def _gin_dense(x, agg, wa, ba, g, be, wb, bb, tanh):
    return pl.pallas_call(
        functools.partial(_gin_dense_body, tanh),
        out_shape=jax.ShapeDtypeStruct((_N, _D), jnp.float32),
    )(x, agg, wa, ba, g, be, wb, bb)


def _final_body(x_ref, agg_ref, wa_ref, ba_ref, g_ref, be_ref, wb_ref,
                bb_ref, batch_ref, wh1_ref, bh1_ref, wh2_ref, bh2_ref,
                o_ref):
    h1 = jnp.tanh(_gin_math(x_ref, agg_ref, wa_ref, ba_ref, g_ref, be_ref,
                            wb_ref, bb_ref))
    seg = lax.broadcasted_iota(jnp.int32, (_N, _G), 1)
    onehot = (batch_ref[...] == seg).astype(jnp.float32)
    sums = lax.dot_general(onehot, h1, (((0,), (0,)), ((), ())),
                           precision=lax.Precision.DEFAULT,
                           preferred_element_type=jnp.float32)
    cnt = jnp.sum(onehot, axis=0)
    pooled = sums / jnp.maximum(cnt, 1.0)[:, None]
    p = jnp.maximum(_dot(pooled, wh1_ref[...]) + bh1_ref[...], 0.0)
    o_ref[...] = _dot(p, wh2_ref[...]) + bh2_ref[...]


def _final(x, agg, wa, ba, g, be, wb, bb, batch2d, wh1, bh1, wh2, bh2):
    return pl.pallas_call(
        _final_body,
        out_shape=jax.ShapeDtypeStruct((_G, wh2.shape[1]), jnp.float32),
    )(x, agg, wa, ba, g, be, wb, bb, batch2d, wh1, bh1, wh2, bh2)


def kernel(x, edge_index, batch, W1a, b1a, g1, be1, W1b, b1b, W2a, b2a, g2,
           be2, W2b, b2b, W3a, b3a, g3, be3, W3b, b3b, Wh1, bh1, Wh2, bh2):
    src4 = edge_index[0].reshape(_NS, _NG, _GC, _CH)
    dst4 = edge_index[1].reshape(_NS, _NG, _GC, _CH)
    batch2d = batch.reshape(_N, 1)

    agg = _edge_agg(x, src4, dst4)
    h = _gin_dense(x, agg, W1a, b1a, g1, be1, W1b, b1b, False)
    agg = _edge_agg(h, src4, dst4)
    h = _gin_dense(h, agg, W2a, b2a, g2, be2, W2b, b2b, True)
    agg = _edge_agg(h, src4, dst4)
    return _final(h, agg, W3a, b3a, g3, be3, W3b, b3b, batch2d,
                  Wh1, bh1, Wh2, bh2)
